# baseline (speedup 1.0000x reference)
import functools

import jax
import jax.numpy as jnp
from jax import lax
from jax.experimental import pallas as pl
from jax.experimental.pallas import tpu as pltpu
from jax.experimental.pallas import tpu_sc as plsc

_NR = 320000
_NROWS = 1026
_NC = 128
_NCG = 4
_NEG = 8
_CW = _NC // _NCG
_RPAD = 1032
_TLEN = _CW * 2 * _RPAD
_EPW = _NR // _NEG
_CHUNK = 800
_NCHUNK = _EPW // _CHUNK


def _body(params_h, table_h, r_h, out_h,
          table_v, params_v, r_v, n_v, h_v, out_v,
          rsem, osem0, osem1):
    cid = lax.axis_index("c")
    sid = lax.axis_index("s")
    wid = sid * 2 + cid
    g = wid % _NCG
    eg = wid // _NCG
    base_row = eg * _EPW

    pltpu.sync_copy(table_h.at[g], table_v)
    pltpu.sync_copy(params_h, params_v)
    d_vec = params_v[0]
    cut_vec = params_v[1]
    iota16 = lax.iota(jnp.int32, 16)

    osems = (osem0, osem1)

    def out_dst(row0):
        return out_h.at[pl.ds(row0, _CHUNK), pl.ds(g * _CW, _CW)]

    pltpu.async_copy(r_h.at[pl.ds(base_row, _CHUNK)], r_v.at[0], rsem)

    def chunk_pair(c2, carry):
        for b in range(2):
            c = c2 * 2 + b
            row0 = base_row + c * _CHUNK
            pltpu.make_async_copy(
                r_h.at[pl.ds(row0, _CHUNK)], r_v.at[b], rsem).wait()

            @pl.when(c + 1 < _NCHUNK)
            def _prefetch():
                pltpu.async_copy(
                    r_h.at[pl.ds(row0 + _CHUNK, _CHUNK)], r_v.at[1 - b], rsem)

            def prep(i, carry2):
                rr = r_v[b, pl.ds(i * 16, 16)]
                x = jnp.minimum(jnp.maximum(rr, 0.0), cut_vec)
                q = x / d_vec
                ni = q.astype(jnp.int32)
                t = (x - ni.astype(jnp.float32) * d_vec) / d_vec
                t2 = t * t
                t3 = t2 * t
                n_v[pl.ds(i * 16, 16)] = ni
                h_v[0, pl.ds(i * 16, 16)] = 2.0 * t3 - 3.0 * t2 + 1.0
                h_v[1, pl.ds(i * 16, 16)] = t3 - 2.0 * t2 + t
                h_v[2, pl.ds(i * 16, 16)] = 3.0 * t2 - 2.0 * t3
                h_v[3, pl.ds(i * 16, 16)] = t3 - t2
                return carry2

            lax.fori_loop(0, _CHUNK // 16, prep, 0)

            @pl.when(c >= 2)
            def _wait_out():
                pltpu.make_async_copy(
                    out_v.at[b], out_dst(row0), osems[b]).wait()

            def elem_block(bb, carry2):
                base = bb * 16
                nv = n_v[pl.ds(base, 16)]
                nv1 = nv + 1
                h0v = h_v[0, pl.ds(base, 16)]
                h1v = h_v[1, pl.ds(base, 16)]
                h2v = h_v[2, pl.ds(base, 16)]
                h3v = h_v[3, pl.ds(base, 16)]
                rowv = iota16 + base
                ob = out_v.at[b]
                for ch in range(_CW):
                    vref = table_v.at[pl.ds((2 * ch) * _RPAD, _RPAD)]
                    dref = table_v.at[pl.ds((2 * ch + 1) * _RPAD, _RPAD)]
                    vn = plsc.load_gather(vref, [nv])
                    vn1 = plsc.load_gather(vref, [nv1])
                    dn = plsc.load_gather(dref, [nv])
                    dn1 = plsc.load_gather(dref, [nv1])
                    acc = h0v * vn + h1v * dn + h2v * vn1 + h3v * dn1
                    colv = jnp.full((16,), ch, jnp.int32)
                    plsc.store_scatter(ob, [rowv, colv], acc)
                return carry2

            lax.fori_loop(0, _CHUNK // 16, elem_block, 0)
            pltpu.async_copy(out_v.at[b], out_dst(row0), osems[b])
        return carry

    lax.fori_loop(0, _NCHUNK // 2, chunk_pair, 0)
    for b in range(2):
        pltpu.make_async_copy(out_v.at[b], out_dst(base_row), osems[b]).wait()


def kernel(r, spline_values, spline_derivatives, spline_spacing, cutoff):
    assert r.shape == (_NR,) and spline_values.shape == (_NROWS, _NC)
    spacing = jnp.asarray(spline_spacing, jnp.float32)
    vt = spline_values.astype(jnp.float32).T
    dt = (spline_derivatives.astype(jnp.float32) * spacing).T
    pad = jnp.zeros((_NC, _RPAD - _NROWS), jnp.float32)
    inter = jnp.stack(
        [jnp.concatenate([vt, pad], axis=1), jnp.concatenate([dt, pad], axis=1)],
        axis=1,
    )
    table = inter.reshape(_NCG, _TLEN)
    params = jnp.stack(
        [
            jnp.full((16,), spacing, jnp.float32),
            jnp.full((16,), jnp.asarray(cutoff, jnp.float32), jnp.float32),
        ]
    )

    mesh = plsc.VectorSubcoreMesh(core_axis_name="c", subcore_axis_name="s")
    run = pl.kernel(
        _body,
        out_type=jax.ShapeDtypeStruct((_NR, _NC), jnp.float32),
        mesh=mesh,
        compiler_params=pltpu.CompilerParams(
            use_tc_tiling_on_sc=False, needs_layout_passes=False
        ),
        scratch_types=[
            pltpu.VMEM((_TLEN,), jnp.float32),
            pltpu.VMEM((2, 16), jnp.float32),
            pltpu.VMEM((2, _CHUNK), jnp.float32),
            pltpu.VMEM((_CHUNK,), jnp.int32),
            pltpu.VMEM((4, _CHUNK), jnp.float32),
            pltpu.VMEM((2, _CHUNK, _CW), jnp.float32),
            pltpu.SemaphoreType.DMA,
            pltpu.SemaphoreType.DMA,
            pltpu.SemaphoreType.DMA,
        ],
    )
    return run(params, table, r.astype(jnp.float32))


# speedup vs baseline: 1.5704x; 1.5704x over previous
import functools

import jax
import jax.numpy as jnp
from jax import lax
from jax.experimental import pallas as pl
from jax.experimental.pallas import tpu as pltpu
from jax.experimental.pallas import tpu_sc as plsc

_NR = 320000
_NROWS = 1026
_NC = 128
_NCG = 4
_NEG = 8
_CW = _NC // _NCG
_RPAD = 1032
_TLEN = _CW * 2 * _RPAD
_EPW = _NR // _NEG
_CHUNK = 800
_NCHUNK = _EPW // _CHUNK


def _body(params_h, table_h, r_h, out_h,
          table_v, params_v, r_v, n_v, h_v, out_v,
          rsem, osem0, osem1):
    cid = lax.axis_index("c")
    sid = lax.axis_index("s")
    wid = sid * 2 + cid
    g = wid % _NCG
    eg = wid // _NCG
    base_row = eg * _EPW

    pltpu.sync_copy(table_h.at[g], table_v)
    pltpu.sync_copy(params_h, params_v)
    d_vec = params_v[0]
    cut_vec = params_v[1]
    iota16 = lax.iota(jnp.int32, 16)

    osems = (osem0, osem1)

    def out_dst(row0):
        return out_h.at[pl.ds(row0, _CHUNK), pl.ds(g * _CW, _CW)]

    pltpu.async_copy(r_h.at[pl.ds(base_row, _CHUNK)], r_v.at[0], rsem)

    def chunk_pair(c2, carry):
        for b in range(2):
            c = c2 * 2 + b
            row0 = base_row + c * _CHUNK
            pltpu.make_async_copy(
                r_h.at[pl.ds(row0, _CHUNK)], r_v.at[b], rsem).wait()

            @pl.when(c + 1 < _NCHUNK)
            def _prefetch():
                pltpu.async_copy(
                    r_h.at[pl.ds(row0 + _CHUNK, _CHUNK)], r_v.at[1 - b], rsem)

            @plsc.parallel_loop(0, _CHUNK // 16, unroll=2)
            def prep(i):
                rr = r_v[b, pl.ds(i * 16, 16)]
                x = jnp.minimum(jnp.maximum(rr, 0.0), cut_vec)
                q = x / d_vec
                ni = q.astype(jnp.int32)
                t = (x - ni.astype(jnp.float32) * d_vec) / d_vec
                t2 = t * t
                t3 = t2 * t
                n_v[pl.ds(i * 16, 16)] = ni
                h_v[0, pl.ds(i * 16, 16)] = 2.0 * t3 - 3.0 * t2 + 1.0
                h_v[1, pl.ds(i * 16, 16)] = t3 - 2.0 * t2 + t
                h_v[2, pl.ds(i * 16, 16)] = 3.0 * t2 - 2.0 * t3
                h_v[3, pl.ds(i * 16, 16)] = t3 - t2

            @pl.when(c >= 2)
            def _wait_out():
                pltpu.make_async_copy(
                    out_v.at[b], out_dst(row0), osems[b]).wait()

            @plsc.parallel_loop(0, _CHUNK // 16, unroll=2)
            def elem_block(bb):
                base = bb * 16
                nv = n_v[pl.ds(base, 16)]
                nv1 = nv + 1
                h0v = h_v[0, pl.ds(base, 16)]
                h1v = h_v[1, pl.ds(base, 16)]
                h2v = h_v[2, pl.ds(base, 16)]
                h3v = h_v[3, pl.ds(base, 16)]
                rowv = iota16 + base
                ob = out_v.at[b]
                for ch in range(_CW):
                    vref = table_v.at[pl.ds((2 * ch) * _RPAD, _RPAD)]
                    dref = table_v.at[pl.ds((2 * ch + 1) * _RPAD, _RPAD)]
                    vn = plsc.load_gather(vref, [nv])
                    vn1 = plsc.load_gather(vref, [nv1])
                    dn = plsc.load_gather(dref, [nv])
                    dn1 = plsc.load_gather(dref, [nv1])
                    acc = h0v * vn + h1v * dn + h2v * vn1 + h3v * dn1
                    colv = jnp.full((16,), ch, jnp.int32)
                    plsc.store_scatter(ob, [rowv, colv], acc)
            pltpu.async_copy(out_v.at[b], out_dst(row0), osems[b])
        return carry

    lax.fori_loop(0, _NCHUNK // 2, chunk_pair, 0)
    for b in range(2):
        pltpu.make_async_copy(out_v.at[b], out_dst(base_row), osems[b]).wait()


def kernel(r, spline_values, spline_derivatives, spline_spacing, cutoff):
    assert r.shape == (_NR,) and spline_values.shape == (_NROWS, _NC)
    spacing = jnp.asarray(spline_spacing, jnp.float32)
    vt = spline_values.astype(jnp.float32).T
    dt = (spline_derivatives.astype(jnp.float32) * spacing).T
    pad = jnp.zeros((_NC, _RPAD - _NROWS), jnp.float32)
    inter = jnp.stack(
        [jnp.concatenate([vt, pad], axis=1), jnp.concatenate([dt, pad], axis=1)],
        axis=1,
    )
    table = inter.reshape(_NCG, _TLEN)
    params = jnp.stack(
        [
            jnp.full((16,), spacing, jnp.float32),
            jnp.full((16,), jnp.asarray(cutoff, jnp.float32), jnp.float32),
        ]
    )

    mesh = plsc.VectorSubcoreMesh(core_axis_name="c", subcore_axis_name="s")
    run = pl.kernel(
        _body,
        out_type=jax.ShapeDtypeStruct((_NR, _NC), jnp.float32),
        mesh=mesh,
        compiler_params=pltpu.CompilerParams(
            use_tc_tiling_on_sc=False, needs_layout_passes=False
        ),
        scratch_types=[
            pltpu.VMEM((_TLEN,), jnp.float32),
            pltpu.VMEM((2, 16), jnp.float32),
            pltpu.VMEM((2, _CHUNK), jnp.float32),
            pltpu.VMEM((_CHUNK,), jnp.int32),
            pltpu.VMEM((4, _CHUNK), jnp.float32),
            pltpu.VMEM((2, _CHUNK, _CW), jnp.float32),
            pltpu.SemaphoreType.DMA,
            pltpu.SemaphoreType.DMA,
            pltpu.SemaphoreType.DMA,
        ],
    )
    return run(params, table, r.astype(jnp.float32))
